# Initial kernel scaffold; baseline (speedup 1.0000x reference)
import jax
import jax.numpy as jnp
from jax.experimental import pallas as pl


def kernel(token_ids, image_feat, token_type_ids, position_ids, W_word, W_pos, W_tt_vis, W_pos_vis, W_proj, b_proj, gamma, beta):
    raise NotImplementedError("write your pallas kernel here")


# speedup vs baseline: 2.0976x; 2.0976x over previous
import functools

import jax
import jax.numpy as jnp
from jax import lax
from jax.experimental import pallas as pl
from jax.experimental.pallas import tpu as pltpu
from jax.experimental.pallas import tpu_sc as plsc

B = 1024
S = 128
NB = 36
H = 768
V = 30522
VD = 2048
EPS = 1e-12

NC = 2
NS = 16
NW = NC * NS

N_TOK = B * S
TOK_PER_W = N_TOK // NW
GCH = 128
N_GCH = TOK_PER_W // GCH

TEXT_BLK = 1024
VIS_BLK = 512
N_VIS = B * NB


def _sc_gather_body(table_hbm, idx_hbm, out_hbm, idx_v, rows_v, sem):
    wid = lax.axis_index("s") * NC + lax.axis_index("c")
    base = wid * TOK_PER_W

    def step(i, carry):
        off = base + i * GCH
        pltpu.sync_copy(idx_hbm.at[pl.ds(off, GCH)], idx_v)
        pltpu.async_copy(table_hbm.at[idx_v], rows_v, sem).wait()
        pltpu.sync_copy(rows_v, out_hbm.at[pl.ds(off, GCH)])
        return carry

    lax.fori_loop(0, N_GCH, step, 0)


def _sc_gather(table, idx_flat):
    mesh = plsc.VectorSubcoreMesh(
        core_axis_name="c", subcore_axis_name="s",
        num_cores=NC, num_subcores=NS)
    k = pl.kernel(
        _sc_gather_body,
        out_type=jax.ShapeDtypeStruct((N_TOK, H), jnp.float32),
        mesh=mesh,
        scratch_types=[
            pltpu.VMEM((GCH,), jnp.int32),
            pltpu.VMEM((GCH, H), jnp.float32),
            pltpu.SemaphoreType.DMA,
        ],
    )
    return k(table, idx_flat)


def _layer_norm_rows(x, gamma, beta):
    mean = jnp.mean(x, axis=-1, keepdims=True)
    var = jnp.mean((x - mean) ** 2, axis=-1, keepdims=True)
    return (x - mean) / jnp.sqrt(var + EPS) * gamma + beta


def _text_body(words_ref, pos_ref, tt_ref, wpos_ref, w01_ref, gamma_ref,
               beta_ref, out_ref):
    words = words_ref[...]
    pos = pos_ref[0, 0, :]
    tt = tt_ref[0, 0, :]
    onehot = (pos[:, None] ==
              lax.broadcasted_iota(jnp.int32, (TEXT_BLK, S), 1))
    posemb = jnp.dot(onehot.astype(jnp.float32), wpos_ref[...],
                     preferred_element_type=jnp.float32)
    w0 = w01_ref[0, :]
    w1 = w01_ref[1, :]
    ttemb = w0[None, :] + tt.astype(jnp.float32)[:, None] * (w1 - w0)[None, :]
    x = words + posemb + ttemb
    out_ref[...] = _layer_norm_rows(x, gamma_ref[...], beta_ref[...])


def _vis_body(img_ref, wproj_ref, row_ref, gamma_ref, beta_ref, out_ref):
    v = jnp.dot(img_ref[...], wproj_ref[...],
                preferred_element_type=jnp.float32)
    x = v + row_ref[0, :][None, :]
    out_ref[...] = _layer_norm_rows(x, gamma_ref[...], beta_ref[...])


def _text_call(words, pos3, tt3, wpos, w01, gamma, beta):
    n_blk = N_TOK // TEXT_BLK
    return pl.pallas_call(
        _text_body,
        grid=(n_blk,),
        in_specs=[
            pl.BlockSpec((TEXT_BLK, H), lambda i: (i, 0)),
            pl.BlockSpec((1, 1, TEXT_BLK), lambda i: (i, 0, 0)),
            pl.BlockSpec((1, 1, TEXT_BLK), lambda i: (i, 0, 0)),
            pl.BlockSpec((S, H), lambda i: (0, 0)),
            pl.BlockSpec((8, H), lambda i: (0, 0)),
            pl.BlockSpec((H,), lambda i: (0,)),
            pl.BlockSpec((H,), lambda i: (0,)),
        ],
        out_specs=pl.BlockSpec((TEXT_BLK, H), lambda i: (i, 0)),
        out_shape=jax.ShapeDtypeStruct((N_TOK, H), jnp.float32),
    )(words, pos3, tt3, wpos, w01, gamma, beta)


def _vis_call(img2d, wprojT, row, gamma, beta):
    n_blk = N_VIS // VIS_BLK
    return pl.pallas_call(
        _vis_body,
        grid=(n_blk,),
        in_specs=[
            pl.BlockSpec((VIS_BLK, VD), lambda i: (i, 0)),
            pl.BlockSpec((VD, H), lambda i: (0, 0)),
            pl.BlockSpec((8, H), lambda i: (0, 0)),
            pl.BlockSpec((H,), lambda i: (0,)),
            pl.BlockSpec((H,), lambda i: (0,)),
        ],
        out_specs=pl.BlockSpec((VIS_BLK, H), lambda i: (i, 0)),
        out_shape=jax.ShapeDtypeStruct((N_VIS, H), jnp.float32),
    )(img2d, wprojT, row, gamma, beta)


def kernel(token_ids, image_feat, token_type_ids, position_ids, W_word,
           W_pos, W_tt_vis, W_pos_vis, W_proj, b_proj, gamma, beta):
    idx_flat = token_ids.reshape(-1).astype(jnp.int32)
    words = _sc_gather(W_word, idx_flat)

    pos3 = position_ids.reshape(N_TOK // TEXT_BLK, 1, TEXT_BLK)
    tt3 = token_type_ids.reshape(N_TOK // TEXT_BLK, 1, TEXT_BLK)
    text = _text_call(words, pos3, tt3, W_pos[:S], W_word[:8], gamma, beta)

    vrow = (b_proj + W_tt_vis[1] + W_pos_vis[0])[None, :]
    vrow8 = jnp.broadcast_to(vrow, (8, H))
    vis = _vis_call(image_feat.reshape(N_VIS, VD), W_proj.T, vrow8,
                    gamma, beta)

    return (text.reshape(B, S, H), vis.reshape(B, NB, H))
